# baseline (speedup 1.0000x reference)
import jax
import jax.numpy as jnp
from jax import lax
from jax.experimental import pallas as pl
from jax.experimental.pallas import tpu as pltpu
from jax.experimental.pallas import tpu_sc as plsc

NC = 2
NS = 16
NW = NC * NS

D = 8
K = 400
SB = 50
NSB = K // SB
GRP = K // 16
IB = 40


def _acc_geom(N):
  nrow = -(-N // NS)
  nrow = -(-nrow // 8) * 8
  npad = nrow * NS
  base = nrow // 8
  d = max(k for k in range(1, 65) if base % k == 0)
  zr = 8 * d
  nz = nrow // zr
  assert zr * nz == nrow and zr % 8 == 0
  return npad, nrow, zr, nz


def _matvec_kernel(N, E):
  EW = E // NW
  NCH = EW // K
  NP, NROW, ZR, NZ = _acc_geom(N)
  QR = K * D * D // 128
  assert EW * NW == E and NCH * K == EW
  assert NCH % 4 == 1 and NCH >= 9

  mesh = plsc.VectorSubcoreMesh(
      core_axis_name="c", subcore_axis_name="s", num_cores=NC,
      num_subcores=NS)

  def body(x_hbm, q2_hbm, row1_hbm, col1_hbm, zeros_hbm, out_hbm,
           acc, colv0, rowv0, colv1, rowv1, colv2, rowv2, colv3, rowv3,
           qv0, qv1, xg0, xg1, xg2, xg3, yv0, yv1, rs0, rs1, obuf,
           sem_i0, sem_i1, sem_i2, sem_i3, sem_q0, sem_q1,
           sem_x0, sem_x1, sem_x2, sem_x3, sem_y0, sem_y1):
    c = lax.axis_index("c")
    s = lax.axis_index("s")
    wid = c * NS + s
    e_base = wid * EW

    colv = (colv0, colv1, colv2, colv3)
    rowv = (rowv0, rowv1, rowv2, rowv3)
    qv = (qv0, qv1)
    xg = (xg0, xg1, xg2, xg3)
    yv = (yv0, yv1)
    rsc = (rs0, rs1)
    sem_i = (sem_i0, sem_i1, sem_i2, sem_i3)
    sem_q = (sem_q0, sem_q1)
    sem_x = (sem_x0, sem_x1, sem_x2, sem_x3)
    sem_y = (sem_y0, sem_y1)

    pltpu.sync_copy(zeros_hbm, obuf)
    for z in range(NZ):
      pltpu.sync_copy(obuf, acc.at[pl.ds(s * NROW + z * ZR, ZR), :])
    plsc.subcore_barrier()

    def issue_idx(g, p4):
      e0 = e_base + g * K
      pltpu.async_copy(col1_hbm.at[pl.ds(e0, K)], colv[p4], sem_i[p4])
      pltpu.async_copy(row1_hbm.at[pl.ds(e0, K)], rowv[p4], sem_i[p4])

    def wait_idx(p4):
      pltpu.make_async_copy(
          col1_hbm.at[pl.ds(0, K)], colv[p4], sem_i[p4]).wait()
      pltpu.make_async_copy(
          row1_hbm.at[pl.ds(0, K)], rowv[p4], sem_i[p4]).wait()

    def issue_q(g, p2):
      pltpu.async_copy(q2_hbm.at[pl.ds((e_base + g * K) * D * D // 128, QR), :],
                       qv[p2], sem_q[p2])

    def wait_q(p2):
      pltpu.make_async_copy(q2_hbm.at[pl.ds(0, QR), :],
                            qv[p2], sem_q[p2]).wait()

    def issue_gather(p4):
      pltpu.async_copy(x_hbm.at[colv[p4]], xg[p4], sem_x[p4])

    def wait_gather(p4):
      pltpu.make_async_copy(x_hbm.at[colv[p4]], xg[p4], sem_x[p4]).wait()

    def copy_rows(p4, p2):
      for m in range(K // 16):
        rsc[p2][pl.ds(m * 16, 16)] = rowv[p4][pl.ds(m * 16, 16)]

    def issue_scatter(p2):
      pltpu.async_copy(yv[p2], acc.at[rsc[p2]], sem_y[p2], add=True)

    def drain_scatter(p2):
      pltpu.make_async_copy(yv[p2], acc.at[rsc[p2]], sem_y[p2]).wait()

    ii = [jnp.full((16,), i, jnp.int32) for i in range(D)]

    def compute_chunk(p4, p2):
      def grp_body(grp):
        ev = lax.iota(jnp.int32, 16) + grp * 16
        ev1 = lax.shift_right_logical(ev, 1)
        evc = lax.shift_left(lax.bitwise_and(ev, 1), 6)
        xjs = [plsc.load_gather(xg[p4], [ev, ii[j]]) for j in range(D)]
        for i in range(D):
          y = plsc.load_gather(qv[p2], [ev1, evc + i * D]) * xjs[0]
          for j in range(1, D):
            y = y + plsc.load_gather(qv[p2], [ev1, evc + (i * D + j)]) * xjs[j]
          plsc.store_scatter(yv[p2], [ev, ii[i]], y)

      pl.loop(0, GRP)(grp_body)

    def chunk_step(g, p4, p2, nxt, idx2, idx3, drain):
      if nxt:
        issue_q(g + 1, 1 - p2)
      if idx2:
        wait_idx((p4 + 2) % 4)
        issue_gather((p4 + 2) % 4)
      wait_q(p2)
      wait_gather(p4)
      if drain:
        drain_scatter(p2)
      copy_rows(p4, p2)
      compute_chunk(p4, p2)
      issue_scatter(p2)
      if idx3:
        issue_idx(g + 3, (p4 + 3) % 4)

    issue_idx(0, 0)
    issue_idx(1, 1)
    issue_idx(2, 2)
    wait_idx(0)
    issue_gather(0)
    wait_idx(1)
    issue_gather(1)
    issue_q(0, 0)
    chunk_step(0, 0, 0, True, True, True, False)
    chunk_step(1, 1, 1, True, True, True, False)

    def quad_body(g):
      for o in range(4):
        chunk_step(g + 2 + o, (2 + o) % 4, o % 2, True, True, True, True)

    pl.loop(0, NCH - 5, step=4)(quad_body)
    f = NCH - 3
    chunk_step(f, f % 4, f % 2, True, True, False, True)
    chunk_step(f + 1, (f + 1) % 4, (f + 1) % 2, True, False, False, True)
    chunk_step(f + 2, (f + 2) % 4, (f + 2) % 2, False, False, False, True)
    drain_scatter((f + 1) % 2)
    drain_scatter((f + 2) % 2)

    plsc.subcore_barrier()
    for z in range(NZ):
      r0 = s * NROW + z * ZR
      pltpu.sync_copy(acc.at[pl.ds(r0, ZR), :], obuf)
      pltpu.sync_copy(obuf, out_hbm.at[c, pl.ds(r0, ZR), :])

  return pl.kernel(
      body,
      out_type=jax.ShapeDtypeStruct((NC, NP, D), jnp.float32),
      mesh=mesh,
      compiler_params=pltpu.CompilerParams(
          use_tc_tiling_on_sc=False, needs_layout_passes=False),
      scratch_types=[
          pltpu.VMEM_SHARED((NP, D), jnp.float32),
          pltpu.VMEM((K,), jnp.int32),
          pltpu.VMEM((K,), jnp.int32),
          pltpu.VMEM((K,), jnp.int32),
          pltpu.VMEM((K,), jnp.int32),
          pltpu.VMEM((K,), jnp.int32),
          pltpu.VMEM((K,), jnp.int32),
          pltpu.VMEM((K,), jnp.int32),
          pltpu.VMEM((K,), jnp.int32),
          pltpu.VMEM((K * D * D // 128, 128), jnp.float32),
          pltpu.VMEM((K * D * D // 128, 128), jnp.float32),
          pltpu.VMEM((K, D), jnp.float32),
          pltpu.VMEM((K, D), jnp.float32),
          pltpu.VMEM((K, D), jnp.float32),
          pltpu.VMEM((K, D), jnp.float32),
          pltpu.VMEM((K, D), jnp.float32),
          pltpu.VMEM((K, D), jnp.float32),
          pltpu.VMEM((K,), jnp.int32),
          pltpu.VMEM((K,), jnp.int32),
          pltpu.VMEM((ZR, D), jnp.float32),
          pltpu.SemaphoreType.DMA,
          pltpu.SemaphoreType.DMA,
          pltpu.SemaphoreType.DMA,
          pltpu.SemaphoreType.DMA,
          pltpu.SemaphoreType.DMA,
          pltpu.SemaphoreType.DMA,
          pltpu.SemaphoreType.DMA,
          pltpu.SemaphoreType.DMA,
          pltpu.SemaphoreType.DMA,
          pltpu.SemaphoreType.DMA,
          pltpu.SemaphoreType.DMA,
          pltpu.SemaphoreType.DMA,
      ],
  )


def _degree_kernel(N, E):
  EW = E // NW
  NIR = EW // SB
  NP, NROW, ZR, NZ = _acc_geom(N)
  assert NIR % IB == 0 and IB % 8 == 0

  mesh = plsc.VectorSubcoreMesh(
      core_axis_name="c", subcore_axis_name="s", num_cores=NC,
      num_subcores=NS)

  def body(row2_hbm, col2_hbm, ones_hbm, zeros_hbm, out_hbm,
           dacc, idxv, onesv, obuf):
    c = lax.axis_index("c")
    s = lax.axis_index("s")
    wid = c * NS + s

    pltpu.sync_copy(zeros_hbm, obuf)
    for z in range(NZ):
      pltpu.sync_copy(obuf, dacc.at[pl.ds(s * NROW + z * ZR, ZR), :])
    pltpu.sync_copy(ones_hbm, onesv)
    plsc.subcore_barrier()

    for src in (row2_hbm, col2_hbm):
      def t_body(t, src=src):
        pltpu.sync_copy(src.at[pl.ds(wid * NIR + t * IB, IB), :], idxv)
        for j in range(IB):
          pltpu.sync_copy(onesv, dacc.at[idxv.at[j]], add=True)

      pl.loop(0, NIR // IB)(t_body)

    plsc.subcore_barrier()
    for z in range(NZ):
      r0 = s * NROW + z * ZR
      pltpu.sync_copy(dacc.at[pl.ds(r0, ZR), :], obuf)
      pltpu.sync_copy(obuf, out_hbm.at[c, pl.ds(r0, ZR), :])

  return pl.kernel(
      body,
      out_type=jax.ShapeDtypeStruct((NC, NP, D), jnp.float32),
      mesh=mesh,
      compiler_params=pltpu.CompilerParams(use_tc_tiling_on_sc=False, needs_layout_passes=False),
      scratch_types=[
          pltpu.VMEM_SHARED((NP, D), jnp.float32),
          pltpu.VMEM((IB, SB), jnp.int32),
          pltpu.VMEM((SB, D), jnp.float32),
          pltpu.VMEM((ZR, D), jnp.float32),
      ],
  )


def _combine_body(p_ref, x_ref, tp_ref, res_ref, d0_ref, d1_ref, a0_ref,
                  a1_ref, tn_ref, ro_ref):
  scale = p_ref[0]
  a = p_ref[1]
  b = p_ref[2]
  w = p_ref[3]
  wx = p_ref[4]
  x = x_ref[...]
  d = d0_ref[...] + d1_ref[...]
  off = a0_ref[...] + a1_ref[...]
  lt = scale * (d * x - off) - x
  tn = a * lt - b * tp_ref[...]
  tn_ref[...] = tn
  ro_ref[...] = res_ref[...] + wx * x + w * tn


def _combine_kernel(N):
  R0 = N * D // (128 * 125)
  BR = 5
  assert R0 * 125 * 128 == N * D and R0 % BR == 0
  blk = lambda: pl.BlockSpec((BR, 125, 128), lambda i: (i, 0, 0))
  return pl.pallas_call(
      _combine_body,
      grid=(R0 // BR,),
      in_specs=[pl.BlockSpec(memory_space=pltpu.SMEM)] + [blk()] * 7,
      out_specs=[blk()] * 2,
      out_shape=[jax.ShapeDtypeStruct((R0, 125, 128), jnp.float32)] * 2,
  )


def kernel(h, Q, edge_index, lambda_max, coeffs):
  N, d_ = h.shape
  E = Q.shape[0]
  order = coeffs.shape[0] - 1
  assert d_ == D and E % (NW * K) == 0
  NP, _, ZR, _nz = _acc_geom(N)
  R0 = N * D // (128 * 125)
  rs = (R0, 125, 128)

  q2 = Q.reshape(E * D * D // 128, 128)
  row1 = edge_index[0]
  col1 = edge_index[1]
  row2 = edge_index[0].reshape(E // SB, SB)
  col2 = edge_index[1].reshape(E // SB, SB)
  zeros = jnp.zeros((ZR, D), jnp.float32)
  ones = jnp.ones((SB, D), jnp.float32)

  matvec = _matvec_kernel(N, E)
  degree = _degree_kernel(N, E)
  combine = _combine_kernel(N)

  dd = degree(row2, col2, ones, zeros)[:, :N].reshape((NC,) + rs)
  scale = (2.0 / (lambda_max + 1e-8)).astype(jnp.float32)[0]
  w = jax.nn.softmax(coeffs.astype(jnp.float32))

  def lstep(x, x2, tp2, res2, a, b, wk, wxk):
    acc = matvec(x, q2, row1, col1, zeros)[:, :N].reshape((NC,) + rs)
    p = jnp.stack([scale, a, b, wk, wxk]).astype(jnp.float32)
    return combine(p, x2, tp2, res2, dd[0], dd[1], acc[0], acc[1])

  h2 = h.reshape(rs)
  zres = jnp.zeros(rs, jnp.float32)
  one = jnp.float32(1.0)
  zero = jnp.float32(0.0)
  two = jnp.float32(2.0)

  tc2, res2 = lstep(h, h2, h2, zres, one, zero, w[1], w[0])
  tp2 = h2
  for k in range(2, order + 1):
    x = tc2.reshape(N, D)
    tn2, res2 = lstep(x, tc2, tp2, res2, two, one, w[k], zero)
    tp2, tc2 = tc2, tn2
  return res2.reshape(N, D)


# speedup vs baseline: 1.0077x; 1.0077x over previous
import jax
import jax.numpy as jnp
from jax import lax
from jax.experimental import pallas as pl
from jax.experimental.pallas import tpu as pltpu
from jax.experimental.pallas import tpu_sc as plsc

NC = 2
NS = 16
NW = NC * NS

D = 8
K = 400
SB = 50
NSB = K // SB
GRP = K // 16
IB = 40


def _acc_geom(N):
  nrow = -(-N // NS)
  nrow = -(-nrow // 8) * 8
  npad = nrow * NS
  base = nrow // 8
  d = max(k for k in range(1, 65) if base % k == 0)
  zr = 8 * d
  nz = nrow // zr
  assert zr * nz == nrow and zr % 8 == 0
  return npad, nrow, zr, nz


def _matvec_kernel(N, E):
  EW = E // NW
  NCH = EW // K
  NP, NROW, ZR, NZ = _acc_geom(N)
  QR = K * D * D // 128
  assert EW * NW == E and NCH * K == EW
  assert NCH % 4 == 1 and NCH >= 9

  mesh = plsc.VectorSubcoreMesh(
      core_axis_name="c", subcore_axis_name="s", num_cores=NC,
      num_subcores=NS)

  def body(x_hbm, q2_hbm, row1_hbm, col1_hbm, zeros_hbm, out_hbm,
           acc, colv0, rowv0, colv1, rowv1, colv2, rowv2, colv3, rowv3,
           qv0, qv1, xg0, xg1, xg2, xg3, yv0, yv1, rs0, rs1, obuf,
           sem_i0, sem_i1, sem_i2, sem_i3, sem_q0, sem_q1,
           sem_x0, sem_x1, sem_x2, sem_x3, sem_y0, sem_y1):
    c = lax.axis_index("c")
    s = lax.axis_index("s")
    wid = c * NS + s
    e_base = wid * EW

    colv = (colv0, colv1, colv2, colv3)
    rowv = (rowv0, rowv1, rowv2, rowv3)
    qv = (qv0, qv1)
    xg = (xg0, xg1, xg2, xg3)
    yv = (yv0, yv1)
    rsc = (rs0, rs1)
    sem_i = (sem_i0, sem_i1, sem_i2, sem_i3)
    sem_q = (sem_q0, sem_q1)
    sem_x = (sem_x0, sem_x1, sem_x2, sem_x3)
    sem_y = (sem_y0, sem_y1)

    pltpu.sync_copy(zeros_hbm, obuf)
    for z in range(NZ):
      pltpu.sync_copy(obuf, acc.at[pl.ds(s * NROW + z * ZR, ZR), :])
    plsc.subcore_barrier()

    def issue_idx(g, p4):
      e0 = e_base + g * K
      pltpu.async_copy(col1_hbm.at[pl.ds(e0, K)], colv[p4], sem_i[p4])
      pltpu.async_copy(row1_hbm.at[pl.ds(e0, K)], rowv[p4], sem_i[p4])

    def wait_idx(p4):
      pltpu.make_async_copy(
          col1_hbm.at[pl.ds(0, K)], colv[p4], sem_i[p4]).wait()
      pltpu.make_async_copy(
          row1_hbm.at[pl.ds(0, K)], rowv[p4], sem_i[p4]).wait()

    def issue_q(g, p2):
      pltpu.async_copy(q2_hbm.at[pl.ds((e_base + g * K) * D * D // 128, QR), :],
                       qv[p2], sem_q[p2])

    def wait_q(p2):
      pltpu.make_async_copy(q2_hbm.at[pl.ds(0, QR), :],
                            qv[p2], sem_q[p2]).wait()

    def issue_gather(p4):
      pltpu.async_copy(x_hbm.at[colv[p4]], xg[p4], sem_x[p4])

    def wait_gather(p4):
      pltpu.make_async_copy(x_hbm.at[colv[p4]], xg[p4], sem_x[p4]).wait()

    def copy_rows(p4, p2):
      for m in range(K // 16):
        rsc[p2][pl.ds(m * 16, 16)] = rowv[p4][pl.ds(m * 16, 16)]

    def issue_scatter(p2):
      pltpu.async_copy(yv[p2], acc.at[rsc[p2]], sem_y[p2], add=True)

    def drain_scatter(p2):
      pltpu.make_async_copy(yv[p2], acc.at[rsc[p2]], sem_y[p2]).wait()

    ii = [jnp.full((16,), i, jnp.int32) for i in range(D)]

    def compute_chunk(p4, p2):
      def grp_body(grp):
        ev = lax.iota(jnp.int32, 16) + grp * 16
        ev1 = lax.shift_right_logical(ev, 1)
        evc = lax.shift_left(lax.bitwise_and(ev, 1), 6)
        xjs = [plsc.load_gather(xg[p4], [ev, ii[j]]) for j in range(D)]
        for i in range(D):
          y = plsc.load_gather(qv[p2], [ev1, evc + i]) * xjs[0]
          for j in range(1, D):
            y = y + plsc.load_gather(qv[p2], [ev1, evc + (j * D + i)]) * xjs[j]
          plsc.store_scatter(yv[p2], [ev, ii[i]], y)

      pl.loop(0, GRP)(grp_body)

    def chunk_step(g, p4, p2, nxt, idx2, idx3, drain):
      if nxt:
        issue_q(g + 1, 1 - p2)
      if idx2:
        wait_idx((p4 + 2) % 4)
        issue_gather((p4 + 2) % 4)
      wait_q(p2)
      wait_gather(p4)
      if drain:
        drain_scatter(p2)
      copy_rows(p4, p2)
      compute_chunk(p4, p2)
      issue_scatter(p2)
      if idx3:
        issue_idx(g + 3, (p4 + 3) % 4)

    issue_idx(0, 0)
    issue_idx(1, 1)
    issue_idx(2, 2)
    wait_idx(0)
    issue_gather(0)
    wait_idx(1)
    issue_gather(1)
    issue_q(0, 0)
    chunk_step(0, 0, 0, True, True, True, False)
    chunk_step(1, 1, 1, True, True, True, False)

    def quad_body(g):
      for o in range(4):
        chunk_step(g + 2 + o, (2 + o) % 4, o % 2, True, True, True, True)

    pl.loop(0, NCH - 5, step=4)(quad_body)
    f = NCH - 3
    chunk_step(f, f % 4, f % 2, True, True, False, True)
    chunk_step(f + 1, (f + 1) % 4, (f + 1) % 2, True, False, False, True)
    chunk_step(f + 2, (f + 2) % 4, (f + 2) % 2, False, False, False, True)
    drain_scatter((f + 1) % 2)
    drain_scatter((f + 2) % 2)

    plsc.subcore_barrier()
    for z in range(NZ):
      r0 = s * NROW + z * ZR
      pltpu.sync_copy(acc.at[pl.ds(r0, ZR), :], obuf)
      pltpu.sync_copy(obuf, out_hbm.at[c, pl.ds(r0, ZR), :])

  return pl.kernel(
      body,
      out_type=jax.ShapeDtypeStruct((NC, NP, D), jnp.float32),
      mesh=mesh,
      compiler_params=pltpu.CompilerParams(
          use_tc_tiling_on_sc=False, needs_layout_passes=False),
      scratch_types=[
          pltpu.VMEM_SHARED((NP, D), jnp.float32),
          pltpu.VMEM((K,), jnp.int32),
          pltpu.VMEM((K,), jnp.int32),
          pltpu.VMEM((K,), jnp.int32),
          pltpu.VMEM((K,), jnp.int32),
          pltpu.VMEM((K,), jnp.int32),
          pltpu.VMEM((K,), jnp.int32),
          pltpu.VMEM((K,), jnp.int32),
          pltpu.VMEM((K,), jnp.int32),
          pltpu.VMEM((K * D * D // 128, 128), jnp.float32),
          pltpu.VMEM((K * D * D // 128, 128), jnp.float32),
          pltpu.VMEM((K, D), jnp.float32),
          pltpu.VMEM((K, D), jnp.float32),
          pltpu.VMEM((K, D), jnp.float32),
          pltpu.VMEM((K, D), jnp.float32),
          pltpu.VMEM((K, D), jnp.float32),
          pltpu.VMEM((K, D), jnp.float32),
          pltpu.VMEM((K,), jnp.int32),
          pltpu.VMEM((K,), jnp.int32),
          pltpu.VMEM((ZR, D), jnp.float32),
          pltpu.SemaphoreType.DMA,
          pltpu.SemaphoreType.DMA,
          pltpu.SemaphoreType.DMA,
          pltpu.SemaphoreType.DMA,
          pltpu.SemaphoreType.DMA,
          pltpu.SemaphoreType.DMA,
          pltpu.SemaphoreType.DMA,
          pltpu.SemaphoreType.DMA,
          pltpu.SemaphoreType.DMA,
          pltpu.SemaphoreType.DMA,
          pltpu.SemaphoreType.DMA,
          pltpu.SemaphoreType.DMA,
      ],
  )


def _degree_kernel(N, E):
  EW = E // NW
  NIR = EW // SB
  NP, NROW, ZR, NZ = _acc_geom(N)
  assert NIR % IB == 0 and IB % 8 == 0

  mesh = plsc.VectorSubcoreMesh(
      core_axis_name="c", subcore_axis_name="s", num_cores=NC,
      num_subcores=NS)

  def body(row2_hbm, col2_hbm, ones_hbm, zeros_hbm, out_hbm,
           dacc, idxv, onesv, obuf):
    c = lax.axis_index("c")
    s = lax.axis_index("s")
    wid = c * NS + s

    pltpu.sync_copy(zeros_hbm, obuf)
    for z in range(NZ):
      pltpu.sync_copy(obuf, dacc.at[pl.ds(s * NROW + z * ZR, ZR), :])
    pltpu.sync_copy(ones_hbm, onesv)
    plsc.subcore_barrier()

    for src in (row2_hbm, col2_hbm):
      def t_body(t, src=src):
        pltpu.sync_copy(src.at[pl.ds(wid * NIR + t * IB, IB), :], idxv)
        for j in range(IB):
          pltpu.sync_copy(onesv, dacc.at[idxv.at[j]], add=True)

      pl.loop(0, NIR // IB)(t_body)

    plsc.subcore_barrier()
    for z in range(NZ):
      r0 = s * NROW + z * ZR
      pltpu.sync_copy(dacc.at[pl.ds(r0, ZR), :], obuf)
      pltpu.sync_copy(obuf, out_hbm.at[c, pl.ds(r0, ZR), :])

  return pl.kernel(
      body,
      out_type=jax.ShapeDtypeStruct((NC, NP, D), jnp.float32),
      mesh=mesh,
      compiler_params=pltpu.CompilerParams(use_tc_tiling_on_sc=False, needs_layout_passes=False),
      scratch_types=[
          pltpu.VMEM_SHARED((NP, D), jnp.float32),
          pltpu.VMEM((IB, SB), jnp.int32),
          pltpu.VMEM((SB, D), jnp.float32),
          pltpu.VMEM((ZR, D), jnp.float32),
      ],
  )


def _combine_body(p_ref, x_ref, tp_ref, res_ref, d0_ref, d1_ref, a0_ref,
                  a1_ref, tn_ref, ro_ref):
  scale = p_ref[0]
  a = p_ref[1]
  b = p_ref[2]
  w = p_ref[3]
  wx = p_ref[4]
  x = x_ref[...]
  d = d0_ref[...] + d1_ref[...]
  off = a0_ref[...] + a1_ref[...]
  lt = scale * (d * x - off) - x
  tn = a * lt - b * tp_ref[...]
  tn_ref[...] = tn
  ro_ref[...] = res_ref[...] + wx * x + w * tn


def _combine_kernel(N):
  R0 = N * D // (128 * 125)
  BR = 5
  assert R0 * 125 * 128 == N * D and R0 % BR == 0
  blk = lambda: pl.BlockSpec((BR, 125, 128), lambda i: (i, 0, 0))
  return pl.pallas_call(
      _combine_body,
      grid=(R0 // BR,),
      in_specs=[pl.BlockSpec(memory_space=pltpu.SMEM)] + [blk()] * 7,
      out_specs=[blk()] * 2,
      out_shape=[jax.ShapeDtypeStruct((R0, 125, 128), jnp.float32)] * 2,
  )


def kernel(h, Q, edge_index, lambda_max, coeffs):
  N, d_ = h.shape
  E = Q.shape[0]
  order = coeffs.shape[0] - 1
  assert d_ == D and E % (NW * K) == 0
  NP, _, ZR, _nz = _acc_geom(N)
  R0 = N * D // (128 * 125)
  rs = (R0, 125, 128)

  q2 = jnp.transpose(Q, (0, 2, 1)).reshape(E * D * D // 128, 128)
  row1 = edge_index[0]
  col1 = edge_index[1]
  row2 = edge_index[0].reshape(E // SB, SB)
  col2 = edge_index[1].reshape(E // SB, SB)
  zeros = jnp.zeros((ZR, D), jnp.float32)
  ones = jnp.ones((SB, D), jnp.float32)

  matvec = _matvec_kernel(N, E)
  degree = _degree_kernel(N, E)
  combine = _combine_kernel(N)

  dd = degree(row2, col2, ones, zeros)[:, :N].reshape((NC,) + rs)
  scale = (2.0 / (lambda_max + 1e-8)).astype(jnp.float32)[0]
  w = jax.nn.softmax(coeffs.astype(jnp.float32))

  def lstep(x, x2, tp2, res2, a, b, wk, wxk):
    acc = matvec(x, q2, row1, col1, zeros)[:, :N].reshape((NC,) + rs)
    p = jnp.stack([scale, a, b, wk, wxk]).astype(jnp.float32)
    return combine(p, x2, tp2, res2, dd[0], dd[1], acc[0], acc[1])

  h2 = h.reshape(rs)
  zres = jnp.zeros(rs, jnp.float32)
  one = jnp.float32(1.0)
  zero = jnp.float32(0.0)
  two = jnp.float32(2.0)

  tc2, res2 = lstep(h, h2, h2, zres, one, zero, w[1], w[0])
  tp2 = h2
  for k in range(2, order + 1):
    x = tc2.reshape(N, D)
    tn2, res2 = lstep(x, tc2, tp2, res2, two, one, w[k], zero)
    tp2, tc2 = tc2, tn2
  return res2.reshape(N, D)


# speedup vs baseline: 3.5466x; 3.5194x over previous
import jax
import jax.numpy as jnp
from jax import lax
from jax.experimental import pallas as pl
from jax.experimental.pallas import tpu as pltpu
from jax.experimental.pallas import tpu_sc as plsc

NC = 2
NS = 16
NW = NC * NS

D = 8
K = 400
SB = 50
NSB = K // SB
GRP = K // 16
IB = 40


def _acc_geom(N):
  nrow = -(-N // NS)
  nrow = -(-nrow // 8) * 8
  npad = nrow * NS
  base = nrow // 8
  d = max(k for k in range(1, 65) if base % k == 0)
  zr = 8 * d
  nz = nrow // zr
  assert zr * nz == nrow and zr % 8 == 0
  return npad, nrow, zr, nz


def _matvec_kernel(N, E):
  EB = E // 128
  CB = 3
  K2 = CB * 128
  EBW = -(-EB // NW)
  NCH = -(-EBW // CB)
  while NCH % 4 != 1 or NCH < 9:
    NCH += 1
  NP, NROW, ZR, NZ = _acc_geom(N)
  GR2 = K2 // 16

  mesh = plsc.VectorSubcoreMesh(
      core_axis_name="c", subcore_axis_name="s", num_cores=NC,
      num_subcores=NS)

  def body(x_hbm, qn_hbm, row1_hbm, col1_hbm, zeros_hbm, out_hbm,
           acc, colv0, rowv0, colv1, rowv1, colv2, rowv2, colv3, rowv3,
           qv0, qv1, xg0, xg1, xg2, xg3, yv0, yv1, rs0, rs1, obuf,
           sem_i0, sem_i1, sem_i2, sem_i3, sem_q0, sem_q1,
           sem_x0, sem_x1, sem_x2, sem_x3, sem_y0, sem_y1):
    c = lax.axis_index("c")
    s = lax.axis_index("s")
    wid = c * NS + s
    eb_base = wid * EBW

    colv = (colv0, colv1, colv2, colv3)
    rowv = (rowv0, rowv1, rowv2, rowv3)
    qv = (qv0, qv1)
    xg = (xg0, xg1, xg2, xg3)
    yv = (yv0, yv1)
    rsc = (rs0, rs1)
    sem_i = (sem_i0, sem_i1, sem_i2, sem_i3)
    sem_q = (sem_q0, sem_q1)
    sem_x = (sem_x0, sem_x1, sem_x2, sem_x3)
    sem_y = (sem_y0, sem_y1)

    pltpu.sync_copy(zeros_hbm, obuf)
    for z in range(NZ):
      pltpu.sync_copy(obuf, acc.at[pl.ds(s * NROW + z * ZR, ZR), :])
    plsc.subcore_barrier()

    def eb0c(g):
      return jnp.minimum(eb_base + g * CB, EB - CB)

    def issue_idx(g, p4):
      e0 = eb0c(g) * 128
      pltpu.async_copy(col1_hbm.at[pl.ds(e0, K2)], colv[p4], sem_i[p4])
      pltpu.async_copy(row1_hbm.at[pl.ds(e0, K2)], rowv[p4], sem_i[p4])

    def wait_idx(p4):
      pltpu.make_async_copy(
          col1_hbm.at[pl.ds(0, K2)], colv[p4], sem_i[p4]).wait()
      pltpu.make_async_copy(
          row1_hbm.at[pl.ds(0, K2)], rowv[p4], sem_i[p4]).wait()

    def issue_q(g, p2):
      pltpu.async_copy(qn_hbm.at[:, pl.ds(eb0c(g), CB), :, :],
                       qv[p2], sem_q[p2])

    def wait_q(p2):
      pltpu.make_async_copy(qn_hbm.at[:, pl.ds(0, CB), :, :],
                            qv[p2], sem_q[p2]).wait()

    def issue_gather(p4):
      pltpu.async_copy(x_hbm.at[colv[p4]], xg[p4], sem_x[p4])

    def wait_gather(p4):
      pltpu.make_async_copy(x_hbm.at[colv[p4]], xg[p4], sem_x[p4]).wait()

    def copy_rows(g, p4, p2):
      for m in range(K2 // 16):
        v = rowv[p4][pl.ds(m * 16, 16)]
        eb_m = eb_base + g * CB + m // 8
        ph = (eb_m >= EB) | (eb_m >= eb_base + EBW)
        rsc[p2][pl.ds(m * 16, 16)] = jnp.where(ph, N, v)

    def issue_scatter(p2):
      pltpu.async_copy(yv[p2], acc.at[rsc[p2]], sem_y[p2], add=True)

    def drain_scatter(p2):
      pltpu.make_async_copy(yv[p2], acc.at[rsc[p2]], sem_y[p2]).wait()

    ii = [jnp.full((16,), i, jnp.int32) for i in range(D)]

    def compute_chunk(p4, p2):
      def grp_body(grp):
        ev = lax.iota(jnp.int32, 16) + grp * 16
        ebo = grp // 8
        el0 = (grp % 8) * 16
        xjs = [plsc.load_gather(xg[p4], [ev, ii[j]]) for j in range(D)]
        for i in range(D):
          y = qv[p2][i, ebo, 0, pl.ds(el0, 16)] * xjs[0]
          for j in range(1, D):
            y = y + qv[p2][i, ebo, j, pl.ds(el0, 16)] * xjs[j]
          plsc.store_scatter(yv[p2], [ev, ii[i]], y)

      pl.loop(0, GR2)(grp_body)

    def chunk_step(g, p4, p2, nxt, idx2, idx3, drain):
      if nxt:
        issue_q(g + 1, 1 - p2)
      if idx2:
        wait_idx((p4 + 2) % 4)
        issue_gather((p4 + 2) % 4)
      wait_q(p2)
      wait_gather(p4)
      if drain:
        drain_scatter(p2)
      copy_rows(g, p4, p2)
      compute_chunk(p4, p2)
      issue_scatter(p2)
      if idx3:
        issue_idx(g + 3, (p4 + 3) % 4)

    issue_idx(0, 0)
    issue_idx(1, 1)
    issue_idx(2, 2)
    wait_idx(0)
    issue_gather(0)
    wait_idx(1)
    issue_gather(1)
    issue_q(0, 0)
    chunk_step(0, 0, 0, True, True, True, False)
    chunk_step(1, 1, 1, True, True, True, False)

    def quad_body(g):
      for o in range(4):
        chunk_step(g + 2 + o, (2 + o) % 4, o % 2, True, True, True, True)

    pl.loop(0, NCH - 5, step=4)(quad_body)
    f = NCH - 3
    chunk_step(f, f % 4, f % 2, True, True, False, True)
    chunk_step(f + 1, (f + 1) % 4, (f + 1) % 2, True, False, False, True)
    chunk_step(f + 2, (f + 2) % 4, (f + 2) % 2, False, False, False, True)
    drain_scatter((f + 1) % 2)
    drain_scatter((f + 2) % 2)

    plsc.subcore_barrier()
    for z in range(NZ):
      r0 = s * NROW + z * ZR
      pltpu.sync_copy(acc.at[pl.ds(r0, ZR), :], obuf)
      pltpu.sync_copy(obuf, out_hbm.at[c, pl.ds(r0, ZR), :])

  return pl.kernel(
      body,
      out_type=jax.ShapeDtypeStruct((NC, NP, D), jnp.float32),
      mesh=mesh,
      compiler_params=pltpu.CompilerParams(
          use_tc_tiling_on_sc=False, needs_layout_passes=False),
      scratch_types=[
          pltpu.VMEM_SHARED((NP, D), jnp.float32),
          pltpu.VMEM((K2,), jnp.int32),
          pltpu.VMEM((K2,), jnp.int32),
          pltpu.VMEM((K2,), jnp.int32),
          pltpu.VMEM((K2,), jnp.int32),
          pltpu.VMEM((K2,), jnp.int32),
          pltpu.VMEM((K2,), jnp.int32),
          pltpu.VMEM((K2,), jnp.int32),
          pltpu.VMEM((K2,), jnp.int32),
          pltpu.VMEM((D, CB, D, 128), jnp.float32),
          pltpu.VMEM((D, CB, D, 128), jnp.float32),
          pltpu.VMEM((K2, D), jnp.float32),
          pltpu.VMEM((K2, D), jnp.float32),
          pltpu.VMEM((K2, D), jnp.float32),
          pltpu.VMEM((K2, D), jnp.float32),
          pltpu.VMEM((K2, D), jnp.float32),
          pltpu.VMEM((K2, D), jnp.float32),
          pltpu.VMEM((K2,), jnp.int32),
          pltpu.VMEM((K2,), jnp.int32),
          pltpu.VMEM((ZR, D), jnp.float32),
          pltpu.SemaphoreType.DMA,
          pltpu.SemaphoreType.DMA,
          pltpu.SemaphoreType.DMA,
          pltpu.SemaphoreType.DMA,
          pltpu.SemaphoreType.DMA,
          pltpu.SemaphoreType.DMA,
          pltpu.SemaphoreType.DMA,
          pltpu.SemaphoreType.DMA,
          pltpu.SemaphoreType.DMA,
          pltpu.SemaphoreType.DMA,
          pltpu.SemaphoreType.DMA,
          pltpu.SemaphoreType.DMA,
      ],
  )

def _degree_kernel(N, E):
  EW = E // NW
  NIR = EW // SB
  NP, NROW, ZR, NZ = _acc_geom(N)
  assert NIR % IB == 0 and IB % 8 == 0

  mesh = plsc.VectorSubcoreMesh(
      core_axis_name="c", subcore_axis_name="s", num_cores=NC,
      num_subcores=NS)

  def body(row2_hbm, col2_hbm, ones_hbm, zeros_hbm, out_hbm,
           dacc, idxv, onesv, obuf):
    c = lax.axis_index("c")
    s = lax.axis_index("s")
    wid = c * NS + s

    pltpu.sync_copy(zeros_hbm, obuf)
    for z in range(NZ):
      pltpu.sync_copy(obuf, dacc.at[pl.ds(s * NROW + z * ZR, ZR), :])
    pltpu.sync_copy(ones_hbm, onesv)
    plsc.subcore_barrier()

    for src in (row2_hbm, col2_hbm):
      def t_body(t, src=src):
        pltpu.sync_copy(src.at[pl.ds(wid * NIR + t * IB, IB), :], idxv)
        for j in range(IB):
          pltpu.sync_copy(onesv, dacc.at[idxv.at[j]], add=True)

      pl.loop(0, NIR // IB)(t_body)

    plsc.subcore_barrier()
    for z in range(NZ):
      r0 = s * NROW + z * ZR
      pltpu.sync_copy(dacc.at[pl.ds(r0, ZR), :], obuf)
      pltpu.sync_copy(obuf, out_hbm.at[c, pl.ds(r0, ZR), :])

  return pl.kernel(
      body,
      out_type=jax.ShapeDtypeStruct((NC, NP, D), jnp.float32),
      mesh=mesh,
      compiler_params=pltpu.CompilerParams(use_tc_tiling_on_sc=False, needs_layout_passes=False),
      scratch_types=[
          pltpu.VMEM_SHARED((NP, D), jnp.float32),
          pltpu.VMEM((IB, SB), jnp.int32),
          pltpu.VMEM((SB, D), jnp.float32),
          pltpu.VMEM((ZR, D), jnp.float32),
      ],
  )


def _combine_body(p_ref, x_ref, tp_ref, res_ref, d0_ref, d1_ref, a0_ref,
                  a1_ref, tn_ref, ro_ref):
  scale = p_ref[0]
  a = p_ref[1]
  b = p_ref[2]
  w = p_ref[3]
  wx = p_ref[4]
  x = x_ref[...]
  d = d0_ref[...] + d1_ref[...]
  off = a0_ref[...] + a1_ref[...]
  lt = scale * (d * x - off) - x
  tn = a * lt - b * tp_ref[...]
  tn_ref[...] = tn
  ro_ref[...] = res_ref[...] + wx * x + w * tn


def _combine_kernel(N):
  R0 = N * D // (128 * 125)
  BR = 5
  assert R0 * 125 * 128 == N * D and R0 % BR == 0
  blk = lambda: pl.BlockSpec((BR, 125, 128), lambda i: (i, 0, 0))
  return pl.pallas_call(
      _combine_body,
      grid=(R0 // BR,),
      in_specs=[pl.BlockSpec(memory_space=pltpu.SMEM)] + [blk()] * 7,
      out_specs=[blk()] * 2,
      out_shape=[jax.ShapeDtypeStruct((R0, 125, 128), jnp.float32)] * 2,
  )


def kernel(h, Q, edge_index, lambda_max, coeffs):
  N, d_ = h.shape
  E = Q.shape[0]
  order = coeffs.shape[0] - 1
  assert d_ == D and E % (NW * K) == 0
  NP, _, ZR, _nz = _acc_geom(N)
  R0 = N * D // (128 * 125)
  rs = (R0, 125, 128)

  qn = jnp.transpose(Q, (1, 2, 0)).reshape(D, D, E // 128, 128)
  qn = jnp.transpose(qn, (0, 2, 1, 3))
  row1 = edge_index[0]
  col1 = edge_index[1]
  row2 = edge_index[0].reshape(E // SB, SB)
  col2 = edge_index[1].reshape(E // SB, SB)
  zeros = jnp.zeros((ZR, D), jnp.float32)
  ones = jnp.ones((SB, D), jnp.float32)

  matvec = _matvec_kernel(N, E)
  degree = _degree_kernel(N, E)
  combine = _combine_kernel(N)

  dd = degree(row2, col2, ones, zeros)[:, :N].reshape((NC,) + rs)
  scale = (2.0 / (lambda_max + 1e-8)).astype(jnp.float32)[0]
  w = jax.nn.softmax(coeffs.astype(jnp.float32))

  def lstep(x, x2, tp2, res2, a, b, wk, wxk):
    acc = matvec(x, qn, row1, col1, zeros)[:, :N].reshape((NC,) + rs)
    p = jnp.stack([scale, a, b, wk, wxk]).astype(jnp.float32)
    return combine(p, x2, tp2, res2, dd[0], dd[1], acc[0], acc[1])

  h2 = h.reshape(rs)
  zres = jnp.zeros(rs, jnp.float32)
  one = jnp.float32(1.0)
  zero = jnp.float32(0.0)
  two = jnp.float32(2.0)

  tc2, res2 = lstep(h, h2, h2, zres, one, zero, w[1], w[0])
  tp2 = h2
  for k in range(2, order + 1):
    x = tc2.reshape(N, D)
    tn2, res2 = lstep(x, tc2, tp2, res2, two, one, w[k], zero)
    tp2, tc2 = tc2, tn2
  return res2.reshape(N, D)
